# baseline (speedup 1.0000x reference)
import functools

import jax
import jax.numpy as jnp
from jax.experimental import pallas as pl

_D_MODEL = 2048
_N_EXP = 64
_K = 8
_P = 2
_SCALE = 10.0
_T = 16384

_BT = 256


def _router_block(h_ref, p_ref, mask_ref, probs_ref, logits_ref):
    h = h_ref[...]
    p = p_ref[...]

    eps = 1e-6
    hn = h / (jnp.sqrt(jnp.sum(h * h, axis=1, keepdims=True)) + eps)
    pn = p / (jnp.sqrt(jnp.sum(p * p, axis=1, keepdims=True)) + eps)

    sims = jax.lax.dot_general(
        hn, pn, (((1,), (1,)), ((), ())),
        preferred_element_type=jnp.float32)

    s0 = sims[:, :_N_EXP]
    s1 = sims[:, _N_EXP:]
    m = jnp.maximum(s0, s1)
    lse = m + jnp.log(jnp.exp(s0 - m) + jnp.exp(s1 - m))
    logits = _SCALE * lse

    mx = jnp.max(logits, axis=1, keepdims=True)
    ex = jnp.exp(logits - mx)
    probs = ex / jnp.sum(ex, axis=1, keepdims=True)

    xe = logits[:, :, None]
    xj = logits[:, None, :]
    je = jax.lax.broadcasted_iota(jnp.int32, (_BT, _N_EXP, _N_EXP), 2)
    ee = jax.lax.broadcasted_iota(jnp.int32, (_BT, _N_EXP, _N_EXP), 1)
    beats = (xj > xe) | ((xj == xe) & (je < ee))
    rank = jnp.sum(beats.astype(jnp.float32), axis=2)

    mask_ref[...] = rank < float(_K)
    probs_ref[...] = probs
    logits_ref[...] = logits


@jax.jit
def kernel(h, prototypes):
    p2 = prototypes.transpose(1, 0, 2).reshape(_P * _N_EXP, _D_MODEL)

    grid = (_T // _BT,)
    mask, probs, logits = pl.pallas_call(
        _router_block,
        grid=grid,
        in_specs=[
            pl.BlockSpec((_BT, _D_MODEL), lambda i: (i, 0)),
            pl.BlockSpec((_P * _N_EXP, _D_MODEL), lambda i: (0, 0)),
        ],
        out_specs=[
            pl.BlockSpec((_BT, _N_EXP), lambda i: (i, 0)),
            pl.BlockSpec((_BT, _N_EXP), lambda i: (i, 0)),
            pl.BlockSpec((_BT, _N_EXP), lambda i: (i, 0)),
        ],
        out_shape=[
            jax.ShapeDtypeStruct((_T, _N_EXP), jnp.bool_),
            jax.ShapeDtypeStruct((_T, _N_EXP), jnp.float32),
            jax.ShapeDtypeStruct((_T, _N_EXP), jnp.float32),
        ],
    )(h, p2)

    return (mask, probs, logits, logits)


# speedup vs baseline: 3.4781x; 3.4781x over previous
import functools

import jax
import jax.numpy as jnp
from jax.experimental import pallas as pl

_D_MODEL = 2048
_N_EXP = 64
_K = 8
_P = 2
_SCALE = 10.0
_T = 16384

_BT = 256


def _router_block(h_ref, p_ref, mask_ref, probs_ref, logits_ref):
    h = h_ref[...]
    p = p_ref[...]

    eps = 1e-6
    hn = h / (jnp.sqrt(jnp.sum(h * h, axis=1, keepdims=True)) + eps)
    pn = p / (jnp.sqrt(jnp.sum(p * p, axis=1, keepdims=True)) + eps)

    sims = jax.lax.dot_general(
        hn, pn, (((1,), (1,)), ((), ())),
        preferred_element_type=jnp.float32)

    s0 = sims[:, :_N_EXP]
    s1 = sims[:, _N_EXP:]
    m = jnp.maximum(s0, s1)
    lse = m + jnp.log(jnp.exp(s0 - m) + jnp.exp(s1 - m))
    logits = _SCALE * lse

    mx = jnp.max(logits, axis=1, keepdims=True)
    ex = jnp.exp(logits - mx)
    probs = ex / jnp.sum(ex, axis=1, keepdims=True)

    neg = jnp.float32(-jnp.inf)
    cur = logits
    for _ in range(_K - 1):
        t = jnp.max(cur, axis=1, keepdims=True)
        cur = jnp.where(cur >= t, neg, cur)
    thresh = jnp.max(cur, axis=1, keepdims=True)

    iota = jax.lax.broadcasted_iota(jnp.int32, (_BT, _N_EXP), 1)
    gt = logits > thresh
    n_gt = jnp.sum(gt.astype(jnp.float32), axis=1, keepdims=True)
    eq = logits == thresh
    cand = jnp.where(eq, iota, _N_EXP)
    jmin = jnp.min(cand, axis=1, keepdims=True)
    mask_ref[...] = gt | (eq & (iota == jmin) & (n_gt < float(_K)))
    probs_ref[...] = probs
    logits_ref[...] = logits


@jax.jit
def kernel(h, prototypes):
    p2 = prototypes.transpose(1, 0, 2).reshape(_P * _N_EXP, _D_MODEL)

    grid = (_T // _BT,)
    mask, probs, logits = pl.pallas_call(
        _router_block,
        grid=grid,
        in_specs=[
            pl.BlockSpec((_BT, _D_MODEL), lambda i: (i, 0)),
            pl.BlockSpec((_P * _N_EXP, _D_MODEL), lambda i: (0, 0)),
        ],
        out_specs=[
            pl.BlockSpec((_BT, _N_EXP), lambda i: (i, 0)),
            pl.BlockSpec((_BT, _N_EXP), lambda i: (i, 0)),
            pl.BlockSpec((_BT, _N_EXP), lambda i: (i, 0)),
        ],
        out_shape=[
            jax.ShapeDtypeStruct((_T, _N_EXP), jnp.bool_),
            jax.ShapeDtypeStruct((_T, _N_EXP), jnp.float32),
            jax.ShapeDtypeStruct((_T, _N_EXP), jnp.float32),
        ],
    )(h, p2)

    return (mask, probs, logits, logits)


# speedup vs baseline: 4.4011x; 1.2654x over previous
import functools

import jax
import jax.numpy as jnp
from jax.experimental import pallas as pl

_D_MODEL = 2048
_N_EXP = 64
_K = 8
_P = 2
_SCALE = 10.0
_T = 16384

_BT = 512


def _proto_norm_block(p_ref, pn_ref):
    p = p_ref[...]
    pn_ref[...] = p / (jnp.sqrt(jnp.sum(p * p, axis=1, keepdims=True)) + 1e-6)


def _router_block(h_ref, pn_ref, mask_ref, probs_ref, logits_ref):
    h = h_ref[...]
    pn = pn_ref[...]

    rnorm = 1.0 / (jnp.sqrt(jnp.sum(h * h, axis=1, keepdims=True)) + 1e-6)
    hn = h * rnorm

    sims = jax.lax.dot_general(
        hn, pn, (((1,), (1,)), ((), ())),
        preferred_element_type=jnp.float32)

    s0 = sims[:, :_N_EXP]
    s1 = sims[:, _N_EXP:]
    m = jnp.maximum(s0, s1)
    q = jnp.exp(s0 - m) + jnp.exp(s1 - m)
    logits = _SCALE * (m + jnp.log(q))

    q2 = q * q
    q4 = q2 * q2
    ex = jnp.exp(_SCALE * m) * (q4 * q4 * q2)
    probs = ex / jnp.sum(ex, axis=1, keepdims=True)

    neg = jnp.float32(-jnp.inf)
    cur = logits
    for _ in range(_K - 1):
        t = jnp.max(cur, axis=1, keepdims=True)
        cur = jnp.where(cur >= t, neg, cur)
    thresh = jnp.max(cur, axis=1, keepdims=True)

    iota = jax.lax.broadcasted_iota(jnp.int32, (_BT, _N_EXP), 1)
    gt = logits > thresh
    n_gt = jnp.sum(gt.astype(jnp.float32), axis=1, keepdims=True)
    eq = logits == thresh
    cand = jnp.where(eq, iota, _N_EXP)
    jmin = jnp.min(cand, axis=1, keepdims=True)
    mask_ref[...] = gt | (eq & (iota == jmin) & (n_gt < float(_K)))
    probs_ref[...] = probs
    logits_ref[...] = logits


@jax.jit
def kernel(h, prototypes):
    p2 = prototypes.transpose(1, 0, 2).reshape(_P * _N_EXP, _D_MODEL)

    pn = pl.pallas_call(
        _proto_norm_block,
        out_shape=jax.ShapeDtypeStruct((_P * _N_EXP, _D_MODEL), jnp.float32),
    )(p2)

    grid = (_T // _BT,)
    mask, probs, logits = pl.pallas_call(
        _router_block,
        grid=grid,
        in_specs=[
            pl.BlockSpec((_BT, _D_MODEL), lambda i: (i, 0)),
            pl.BlockSpec((_P * _N_EXP, _D_MODEL), lambda i: (0, 0)),
        ],
        out_specs=[
            pl.BlockSpec((_BT, _N_EXP), lambda i: (i, 0)),
            pl.BlockSpec((_BT, _N_EXP), lambda i: (i, 0)),
            pl.BlockSpec((_BT, _N_EXP), lambda i: (i, 0)),
        ],
        out_shape=[
            jax.ShapeDtypeStruct((_T, _N_EXP), jnp.bool_),
            jax.ShapeDtypeStruct((_T, _N_EXP), jnp.float32),
            jax.ShapeDtypeStruct((_T, _N_EXP), jnp.float32),
        ],
    )(h, pn)

    return (mask, probs, logits, logits)


# speedup vs baseline: 4.4032x; 1.0005x over previous
import functools

import jax
import jax.numpy as jnp
from jax.experimental import pallas as pl

_D_MODEL = 2048
_N_EXP = 64
_K = 8
_P = 2
_SCALE = 10.0
_T = 16384

_BT = 1024


def _proto_norm_block(p_ref, pn_ref):
    p = p_ref[...]
    pn_ref[...] = p / (jnp.sqrt(jnp.sum(p * p, axis=1, keepdims=True)) + 1e-6)


def _router_block(h_ref, pn_ref, mask_ref, probs_ref, logits_ref):
    h = h_ref[...]
    pn = pn_ref[...]

    rnorm = 1.0 / (jnp.sqrt(jnp.sum(h * h, axis=1, keepdims=True)) + 1e-6)
    hn = h * rnorm

    sims = jax.lax.dot_general(
        hn, pn, (((1,), (1,)), ((), ())),
        preferred_element_type=jnp.float32)

    s0 = sims[:, :_N_EXP]
    s1 = sims[:, _N_EXP:]
    m = jnp.maximum(s0, s1)
    q = jnp.exp(s0 - m) + jnp.exp(s1 - m)
    logits = _SCALE * (m + jnp.log(q))

    q2 = q * q
    q4 = q2 * q2
    ex = jnp.exp(_SCALE * m) * (q4 * q4 * q2)
    probs = ex / jnp.sum(ex, axis=1, keepdims=True)

    neg = jnp.float32(-jnp.inf)
    cur = logits
    for _ in range(_K - 1):
        t = jnp.max(cur, axis=1, keepdims=True)
        cur = jnp.where(cur >= t, neg, cur)
    thresh = jnp.max(cur, axis=1, keepdims=True)

    iota = jax.lax.broadcasted_iota(jnp.int32, (_BT, _N_EXP), 1)
    gt = logits > thresh
    n_gt = jnp.sum(gt.astype(jnp.float32), axis=1, keepdims=True)
    eq = logits == thresh
    cand = jnp.where(eq, iota, _N_EXP)
    jmin = jnp.min(cand, axis=1, keepdims=True)
    mask_ref[...] = gt | (eq & (iota == jmin) & (n_gt < float(_K)))
    probs_ref[...] = probs
    logits_ref[...] = logits


@jax.jit
def kernel(h, prototypes):
    p2 = prototypes.transpose(1, 0, 2).reshape(_P * _N_EXP, _D_MODEL)

    pn = pl.pallas_call(
        _proto_norm_block,
        out_shape=jax.ShapeDtypeStruct((_P * _N_EXP, _D_MODEL), jnp.float32),
    )(p2)

    grid = (_T // _BT,)
    mask, probs, logits = pl.pallas_call(
        _router_block,
        grid=grid,
        in_specs=[
            pl.BlockSpec((_BT, _D_MODEL), lambda i: (i, 0)),
            pl.BlockSpec((_P * _N_EXP, _D_MODEL), lambda i: (0, 0)),
        ],
        out_specs=[
            pl.BlockSpec((_BT, _N_EXP), lambda i: (i, 0)),
            pl.BlockSpec((_BT, _N_EXP), lambda i: (i, 0)),
            pl.BlockSpec((_BT, _N_EXP), lambda i: (i, 0)),
        ],
        out_shape=[
            jax.ShapeDtypeStruct((_T, _N_EXP), jnp.bool_),
            jax.ShapeDtypeStruct((_T, _N_EXP), jnp.float32),
            jax.ShapeDtypeStruct((_T, _N_EXP), jnp.float32),
        ],
    )(h, pn)

    return (mask, probs, logits, logits)


# speedup vs baseline: 4.8695x; 1.1059x over previous
import functools

import jax
import jax.numpy as jnp
from jax.experimental import pallas as pl
from jax.experimental.pallas import tpu as pltpu

_D_MODEL = 2048
_N_EXP = 64
_K = 8
_P = 2
_SCALE = 10.0
_T = 16384

_BT = 512
_NB = _T // _BT


def _proto_norm_block(p_ref, pn_ref):
    p = p_ref[...]
    pn_ref[...] = p / (jnp.sqrt(jnp.sum(p * p, axis=1, keepdims=True)) + 1e-6)


def _router_block(h_ref, pn_ref, mask_ref, probs_ref, logits_ref, sims_ref):
    sims = sims_ref[...]

    s0 = sims[:, :_N_EXP]
    s1 = sims[:, _N_EXP:]
    m = jnp.maximum(s0, s1)
    q = jnp.exp(s0 - m) + jnp.exp(s1 - m)
    logits = _SCALE * (m + jnp.log(q))

    q2 = q * q
    q4 = q2 * q2
    ex = jnp.exp(_SCALE * m) * (q4 * q4 * q2)
    probs = ex / jnp.sum(ex, axis=1, keepdims=True)

    neg = jnp.float32(-jnp.inf)
    cur = logits
    for _ in range(_K - 1):
        t = jnp.max(cur, axis=1, keepdims=True)
        cur = jnp.where(cur >= t, neg, cur)
    thresh = jnp.max(cur, axis=1, keepdims=True)

    iota = jax.lax.broadcasted_iota(jnp.int32, (_BT, _N_EXP), 1)
    gt = logits > thresh
    n_gt = jnp.sum(gt.astype(jnp.float32), axis=1, keepdims=True)
    eq = logits == thresh
    cand = jnp.where(eq, iota, _N_EXP)
    jmin = jnp.min(cand, axis=1, keepdims=True)
    mask_ref[...] = gt | (eq & (iota == jmin) & (n_gt < float(_K)))
    probs_ref[...] = probs
    logits_ref[...] = logits

    h = h_ref[...]
    pn = pn_ref[...]

    rnorm = 1.0 / (jnp.sqrt(jnp.sum(h * h, axis=1, keepdims=True)) + 1e-6)
    hn = h * rnorm

    sims_ref[...] = jax.lax.dot_general(
        hn, pn, (((1,), (1,)), ((), ())),
        preferred_element_type=jnp.float32)


@jax.jit
def kernel(h, prototypes):
    p2 = prototypes.transpose(1, 0, 2).reshape(_P * _N_EXP, _D_MODEL)

    pn = pl.pallas_call(
        _proto_norm_block,
        out_shape=jax.ShapeDtypeStruct((_P * _N_EXP, _D_MODEL), jnp.float32),
    )(p2)

    grid = (_NB + 1,)
    out_idx = lambda i: (jnp.maximum(i - 1, 0), 0)
    mask, probs, logits = pl.pallas_call(
        _router_block,
        grid=grid,
        in_specs=[
            pl.BlockSpec((_BT, _D_MODEL), lambda i: (jnp.minimum(i, _NB - 1), 0)),
            pl.BlockSpec((_P * _N_EXP, _D_MODEL), lambda i: (0, 0)),
        ],
        out_specs=[
            pl.BlockSpec((_BT, _N_EXP), out_idx),
            pl.BlockSpec((_BT, _N_EXP), out_idx),
            pl.BlockSpec((_BT, _N_EXP), out_idx),
        ],
        out_shape=[
            jax.ShapeDtypeStruct((_T, _N_EXP), jnp.bool_),
            jax.ShapeDtypeStruct((_T, _N_EXP), jnp.float32),
            jax.ShapeDtypeStruct((_T, _N_EXP), jnp.float32),
        ],
        scratch_shapes=[pltpu.VMEM((_BT, _P * _N_EXP), jnp.float32)],
    )(h, pn)

    return (mask, probs, logits, logits)


# speedup vs baseline: 5.2943x; 1.0872x over previous
import functools

import jax
import jax.numpy as jnp
from jax.experimental import pallas as pl
from jax.experimental.pallas import tpu as pltpu

_D_MODEL = 2048
_N_EXP = 64
_K = 8
_P = 2
_SCALE = 10.0
_T = 16384

_BT = 1024
_NB = _T // _BT


def _proto_norm_block(p_ref, pn_ref):
    p = p_ref[...]
    pn_ref[...] = p / (jnp.sqrt(jnp.sum(p * p, axis=1, keepdims=True)) + 1e-6)


def _router_block(h_ref, pn_ref, mask_ref, probs_ref, logits_ref, sims_ref):
    sims = sims_ref[...]

    s0 = sims[:, :_N_EXP]
    s1 = sims[:, _N_EXP:]
    m = jnp.maximum(s0, s1)
    q = jnp.exp(s0 - m) + jnp.exp(s1 - m)
    logits = _SCALE * (m + jnp.log(q))

    q2 = q * q
    q4 = q2 * q2
    ex = jnp.exp(_SCALE * m) * (q4 * q4 * q2)
    probs = ex / jnp.sum(ex, axis=1, keepdims=True)

    neg = jnp.float32(-jnp.inf)
    cur = logits
    for _ in range(_K - 1):
        t = jnp.max(cur, axis=1, keepdims=True)
        cur = jnp.where(cur >= t, neg, cur)
    thresh = jnp.max(cur, axis=1, keepdims=True)

    iota = jax.lax.broadcasted_iota(jnp.int32, (_BT, _N_EXP), 1)
    gt = logits > thresh
    n_gt = jnp.sum(gt.astype(jnp.float32), axis=1, keepdims=True)
    eq = logits == thresh
    cand = jnp.where(eq, iota, _N_EXP)
    jmin = jnp.min(cand, axis=1, keepdims=True)
    mask_ref[...] = gt | (eq & (iota == jmin) & (n_gt < float(_K)))
    probs_ref[...] = probs
    logits_ref[...] = logits

    h = h_ref[...]
    pn = pn_ref[...]

    rnorm = 1.0 / (jnp.sqrt(jnp.sum(h * h, axis=1, keepdims=True)) + 1e-6)
    hn = h * rnorm

    sims_ref[...] = jax.lax.dot_general(
        hn, pn, (((1,), (1,)), ((), ())),
        preferred_element_type=jnp.float32)


@jax.jit
def kernel(h, prototypes):
    p2 = prototypes.transpose(1, 0, 2).reshape(_P * _N_EXP, _D_MODEL)

    pn = pl.pallas_call(
        _proto_norm_block,
        out_shape=jax.ShapeDtypeStruct((_P * _N_EXP, _D_MODEL), jnp.float32),
    )(p2)

    grid = (_NB + 1,)
    out_idx = lambda i: (jnp.maximum(i - 1, 0), 0)
    mask, probs, logits = pl.pallas_call(
        _router_block,
        grid=grid,
        in_specs=[
            pl.BlockSpec((_BT, _D_MODEL), lambda i: (jnp.minimum(i, _NB - 1), 0)),
            pl.BlockSpec((_P * _N_EXP, _D_MODEL), lambda i: (0, 0)),
        ],
        out_specs=[
            pl.BlockSpec((_BT, _N_EXP), out_idx),
            pl.BlockSpec((_BT, _N_EXP), out_idx),
            pl.BlockSpec((_BT, _N_EXP), out_idx),
        ],
        out_shape=[
            jax.ShapeDtypeStruct((_T, _N_EXP), jnp.bool_),
            jax.ShapeDtypeStruct((_T, _N_EXP), jnp.float32),
            jax.ShapeDtypeStruct((_T, _N_EXP), jnp.float32),
        ],
        scratch_shapes=[pltpu.VMEM((_BT, _P * _N_EXP), jnp.float32)],
    )(h, pn)

    return (mask, probs, logits, logits)
